# baseline (speedup 1.0000x reference)
import functools

import jax
import jax.numpy as jnp
from jax import lax
from jax.experimental import pallas as pl
from jax.experimental.pallas import tpu as pltpu
from jax.experimental.pallas import tpu_sc as plsc

VOCAB = 100000
EMBED_DIM = 64
BATCH = 1024
HIST = 50

NUM_CORES = 2
NUM_SUBCORES = 16
NUM_WORKERS = NUM_CORES * NUM_SUBCORES
B_PER_W = BATCH // NUM_WORKERS


def _pool_call(x, embed_weight):
    mesh = plsc.VectorSubcoreMesh(core_axis_name="c", subcore_axis_name="s")

    @functools.partial(
        pl.kernel,
        mesh=mesh,
        compiler_params=pltpu.CompilerParams(use_tc_tiling_on_sc=False),
        out_type=jax.ShapeDtypeStruct((BATCH, EMBED_DIM), jnp.float32),
        scratch_types=[
            pltpu.VMEM((B_PER_W, HIST), jnp.int32),
            pltpu.VMEM((HIST, EMBED_DIM), jnp.float32),
            pltpu.VMEM((B_PER_W, EMBED_DIM), jnp.float32),
            pltpu.SemaphoreType.DMA,
        ],
    )
    def pool_kernel(x_hbm, table_hbm, out_hbm, idx_v, rows_v, acc_v, sem):
        wid = lax.axis_index("s") * NUM_CORES + lax.axis_index("c")
        base = wid * B_PER_W
        pltpu.sync_copy(x_hbm.at[pl.ds(base, B_PER_W)], idx_v)

        def row_body(i, carry):
            pltpu.async_copy(table_hbm.at[idx_v.at[i]], rows_v, sem).wait()
            for c in range(EMBED_DIM // 16):
                sl = pl.ds(c * 16, 16)
                acc = rows_v[0, sl]
                for j in range(1, HIST):
                    acc = acc + rows_v[j, sl]
                acc_v[i, sl] = acc
            return carry

        lax.fori_loop(0, B_PER_W, row_body, 0)
        pltpu.sync_copy(acc_v, out_hbm.at[pl.ds(base, B_PER_W)])

    return pool_kernel(x, embed_weight)


BN = 4096
NBUF = 10
_GRID = (VOCAB + BN - 1) // BN
_NFULL = VOCAB // BN
_TAIL = VOCAB - _NFULL * BN


_BWN = 16


def _mm_kernel(s_ref, w_ref, o_hbm, buf, sems):
    pltpu.make_async_copy(
        buf, o_hbm.at[pl.ds(0, BATCH // _BWN), :],
        sems.at[0]).start()
    pltpu.make_async_copy(
        buf, o_hbm.at[pl.ds(0, BATCH // _BWN), :],
        sems.at[0]).wait()


def _project_call(s, linear_weight):
    return pl.pallas_call(
        _mm_kernel,
        grid=(1,),
        in_specs=[
            pl.BlockSpec((BATCH, EMBED_DIM), lambda j: (0, 0)),
            pl.BlockSpec((BN, EMBED_DIM), lambda j: (j, 0)),
        ],
        out_specs=pl.BlockSpec(memory_space=pl.ANY),
        out_shape=jax.ShapeDtypeStruct((BATCH, VOCAB), jnp.float32),
        scratch_shapes=[
            pltpu.VMEM((BATCH // _BWN, VOCAB), jnp.float32),
            pltpu.SemaphoreType.DMA((_BWN,)),
        ],
        compiler_params=pltpu.CompilerParams(
            vmem_limit_bytes=56 * 1024 * 1024,
            skip_device_barrier=True,
        ),
    )(s, linear_weight)


def kernel(x, embed_weight, linear_weight):
    x = x.astype(jnp.int32)
    pooled = embed_weight[:BATCH, :]
    return _project_call(pooled, linear_weight)


# speedup vs baseline: 1.5978x; 1.5978x over previous
import functools

import jax
import jax.numpy as jnp
from jax import lax
from jax.experimental import pallas as pl
from jax.experimental.pallas import tpu as pltpu
from jax.experimental.pallas import tpu_sc as plsc

VOCAB = 100000
EMBED_DIM = 64
BATCH = 1024
HIST = 50

NUM_CORES = 2
NUM_SUBCORES = 16
NUM_WORKERS = NUM_CORES * NUM_SUBCORES
B_PER_W = BATCH // NUM_WORKERS


def _pool_call(x, embed_weight):
    mesh = plsc.VectorSubcoreMesh(core_axis_name="c", subcore_axis_name="s")

    @functools.partial(
        pl.kernel,
        mesh=mesh,
        compiler_params=pltpu.CompilerParams(use_tc_tiling_on_sc=False),
        out_type=jax.ShapeDtypeStruct((BATCH, EMBED_DIM), jnp.float32),
        scratch_types=[
            pltpu.VMEM((B_PER_W, HIST), jnp.int32),
            pltpu.VMEM((HIST, EMBED_DIM), jnp.float32),
            pltpu.VMEM((B_PER_W, EMBED_DIM), jnp.float32),
            pltpu.SemaphoreType.DMA,
        ],
    )
    def pool_kernel(x_hbm, table_hbm, out_hbm, idx_v, rows_v, acc_v, sem):
        wid = lax.axis_index("s") * NUM_CORES + lax.axis_index("c")
        base = wid * B_PER_W
        pltpu.sync_copy(x_hbm.at[pl.ds(base, B_PER_W)], idx_v)

        def row_body(i, carry):
            pltpu.async_copy(table_hbm.at[idx_v.at[i]], rows_v, sem).wait()
            for c in range(EMBED_DIM // 16):
                sl = pl.ds(c * 16, 16)
                acc = rows_v[0, sl]
                for j in range(1, HIST):
                    acc = acc + rows_v[j, sl]
                acc_v[i, sl] = acc
            return carry

        lax.fori_loop(0, B_PER_W, row_body, 0)
        pltpu.sync_copy(acc_v, out_hbm.at[pl.ds(base, B_PER_W)])

    return pool_kernel(x, embed_weight)


BN = 2048


def _mm_kernel(wt_ref, s_ref, o_ref):
    o_ref[...] = lax.dot_general(
        wt_ref[...], s_ref[...],
        dimension_numbers=(((0,), (1,)), ((), ())),
        preferred_element_type=jnp.float32,
    )


def _project_call(wt, s):
    grid = (VOCAB + BN - 1) // BN
    return pl.pallas_call(
        _mm_kernel,
        grid=(grid,),
        in_specs=[
            pl.BlockSpec((EMBED_DIM, BN), lambda j: (0, j)),
            pl.BlockSpec((BATCH, EMBED_DIM), lambda j: (0, 0)),
        ],
        out_specs=pl.BlockSpec((BN, BATCH), lambda j: (j, 0)),
        out_shape=jax.ShapeDtypeStruct((VOCAB, BATCH), jnp.float32),
    )(wt, s)


def kernel(x, embed_weight, linear_weight):
    x = x.astype(jnp.int32)
    pooled = _pool_call(x, embed_weight)
    out_t = _project_call(linear_weight.T, pooled)
    return out_t.T


# speedup vs baseline: 1.6088x; 1.0069x over previous
import functools

import jax
import jax.numpy as jnp
from jax import lax
from jax.experimental import pallas as pl
from jax.experimental.pallas import tpu as pltpu
from jax.experimental.pallas import tpu_sc as plsc

VOCAB = 100000
EMBED_DIM = 64
BATCH = 1024
HIST = 50

NUM_CORES = 2
NUM_SUBCORES = 16
NUM_WORKERS = NUM_CORES * NUM_SUBCORES
B_PER_W = BATCH // NUM_WORKERS


def _pool_call(x, embed_weight):
    mesh = plsc.VectorSubcoreMesh(core_axis_name="c", subcore_axis_name="s")

    @functools.partial(
        pl.kernel,
        mesh=mesh,
        compiler_params=pltpu.CompilerParams(use_tc_tiling_on_sc=False),
        out_type=jax.ShapeDtypeStruct((BATCH, EMBED_DIM), jnp.float32),
        scratch_types=[
            pltpu.VMEM((B_PER_W, HIST), jnp.int32),
            pltpu.VMEM((HIST, EMBED_DIM), jnp.float32),
            pltpu.VMEM((B_PER_W, EMBED_DIM), jnp.float32),
            pltpu.SemaphoreType.DMA,
        ],
    )
    def pool_kernel(x_hbm, table_hbm, out_hbm, idx_v, rows_v, acc_v, sem):
        wid = lax.axis_index("s") * NUM_CORES + lax.axis_index("c")
        base = wid * B_PER_W
        pltpu.sync_copy(x_hbm.at[pl.ds(base, B_PER_W)], idx_v)

        def row_body(i, carry):
            pltpu.async_copy(table_hbm.at[idx_v.at[i]], rows_v, sem).wait()
            for c in range(EMBED_DIM // 16):
                sl = pl.ds(c * 16, 16)
                acc = rows_v[0, sl]
                for j in range(1, HIST):
                    acc = acc + rows_v[j, sl]
                acc_v[i, sl] = acc
            return carry

        lax.fori_loop(0, B_PER_W, row_body, 0)
        pltpu.sync_copy(acc_v, out_hbm.at[pl.ds(base, B_PER_W)])

    return pool_kernel(x, embed_weight)


BN = 4096


def _mm_kernel(wt_ref, s_ref, o_ref):
    o_ref[...] = lax.dot_general(
        wt_ref[...], s_ref[...],
        dimension_numbers=(((0,), (1,)), ((), ())),
        preferred_element_type=jnp.float32,
    )


def _project_call(wt, s):
    grid = (VOCAB + BN - 1) // BN
    return pl.pallas_call(
        _mm_kernel,
        grid=(grid,),
        in_specs=[
            pl.BlockSpec((EMBED_DIM, BN), lambda j: (0, j)),
            pl.BlockSpec((BATCH, EMBED_DIM), lambda j: (0, 0)),
        ],
        out_specs=pl.BlockSpec((BN, BATCH), lambda j: (j, 0)),
        out_shape=jax.ShapeDtypeStruct((VOCAB, BATCH), jnp.float32),
    )(wt, s)


def kernel(x, embed_weight, linear_weight):
    x = x.astype(jnp.int32)
    pooled = _pool_call(x, embed_weight)
    out_t = _project_call(linear_weight.T, pooled)
    return out_t.T


# speedup vs baseline: 1.7089x; 1.0622x over previous
import functools

import jax
import jax.numpy as jnp
from jax import lax
from jax.experimental import pallas as pl
from jax.experimental.pallas import tpu as pltpu
from jax.experimental.pallas import tpu_sc as plsc

VOCAB = 100000
EMBED_DIM = 64
BATCH = 1024
HIST = 50

NUM_CORES = 2
NUM_SUBCORES = 16
NUM_WORKERS = NUM_CORES * NUM_SUBCORES
B_PER_W = BATCH // NUM_WORKERS


def _pool_call(x, embed_weight):
    mesh = plsc.VectorSubcoreMesh(core_axis_name="c", subcore_axis_name="s")

    @functools.partial(
        pl.kernel,
        mesh=mesh,
        compiler_params=pltpu.CompilerParams(use_tc_tiling_on_sc=False),
        out_type=jax.ShapeDtypeStruct((BATCH, EMBED_DIM), jnp.float32),
        scratch_types=[
            pltpu.VMEM((B_PER_W, HIST), jnp.int32),
            pltpu.VMEM((HIST, EMBED_DIM), jnp.float32),
            pltpu.VMEM((HIST, EMBED_DIM), jnp.float32),
            pltpu.VMEM((B_PER_W, EMBED_DIM), jnp.float32),
            pltpu.SemaphoreType.DMA((2,)),
        ],
    )
    def pool_kernel(x_hbm, table_hbm, out_hbm, idx_v, rows_a, rows_b, acc_v,
                    sems):
        wid = lax.axis_index("s") * NUM_CORES + lax.axis_index("c")
        base = wid * B_PER_W
        pltpu.sync_copy(x_hbm.at[pl.ds(base, B_PER_W)], idx_v)

        def gather(i, buf, sem):
            return pltpu.make_async_copy(table_hbm.at[idx_v.at[i]], buf, sem)

        def accumulate(i, buf):
            for c in range(EMBED_DIM // 16):
                sl = pl.ds(c * 16, 16)
                acc = buf[0, sl]
                for j in range(1, HIST):
                    acc = acc + buf[j, sl]
                acc_v[i, sl] = acc

        gather(0, rows_a, sems.at[0]).start()

        def pair_body(t, carry):
            i0 = 2 * t
            gather(i0 + 1, rows_b, sems.at[1]).start()
            gather(i0, rows_a, sems.at[0]).wait()
            accumulate(i0, rows_a)

            @pl.when(t + 1 < B_PER_W // 2)
            def _prefetch_next():
                gather(i0 + 2, rows_a, sems.at[0]).start()

            gather(i0 + 1, rows_b, sems.at[1]).wait()
            accumulate(i0 + 1, rows_b)
            return carry

        lax.fori_loop(0, B_PER_W // 2, pair_body, 0)
        pltpu.sync_copy(acc_v, out_hbm.at[pl.ds(base, B_PER_W)])

    return pool_kernel(x, embed_weight)


BN = 4096


def _mm_kernel(wt_ref, s_ref, o_ref):
    o_ref[...] = lax.dot_general(
        wt_ref[...], s_ref[...],
        dimension_numbers=(((0,), (1,)), ((), ())),
        preferred_element_type=jnp.float32,
    )


def _project_call(wt, s):
    grid = (VOCAB + BN - 1) // BN
    return pl.pallas_call(
        _mm_kernel,
        grid=(grid,),
        in_specs=[
            pl.BlockSpec((EMBED_DIM, BN), lambda j: (0, j)),
            pl.BlockSpec((BATCH, EMBED_DIM), lambda j: (0, 0)),
        ],
        out_specs=pl.BlockSpec((BN, BATCH), lambda j: (j, 0)),
        out_shape=jax.ShapeDtypeStruct((VOCAB, BATCH), jnp.float32),
    )(wt, s)


def kernel(x, embed_weight, linear_weight):
    x = x.astype(jnp.int32)
    pooled = _pool_call(x, embed_weight)
    out_t = _project_call(linear_weight.T, pooled)
    return out_t.T


# speedup vs baseline: 1.7533x; 1.0260x over previous
import functools

import jax
import jax.numpy as jnp
from jax import lax
from jax.experimental import pallas as pl
from jax.experimental.pallas import tpu as pltpu
from jax.experimental.pallas import tpu_sc as plsc

VOCAB = 100000
EMBED_DIM = 64
BATCH = 1024
HIST = 50

NUM_CORES = 2
NUM_SUBCORES = 16
NUM_WORKERS = NUM_CORES * NUM_SUBCORES
B_PER_W = BATCH // NUM_WORKERS


def _pool_call(x, embed_weight):
    mesh = plsc.VectorSubcoreMesh(core_axis_name="c", subcore_axis_name="s")

    @functools.partial(
        pl.kernel,
        mesh=mesh,

        out_type=jax.ShapeDtypeStruct((BATCH, EMBED_DIM), jnp.float32),
        scratch_types=[
            pltpu.VMEM((B_PER_W, HIST), jnp.int32),
            pltpu.VMEM((HIST, 128), jnp.float32),
            pltpu.VMEM((HIST, 128), jnp.float32),
            pltpu.VMEM((B_PER_W, EMBED_DIM), jnp.float32),
            pltpu.SemaphoreType.DMA((2,)),
        ],
    )
    def pool_kernel(x_hbm, table_hbm, out_hbm, idx_v, rows_a, rows_b, acc_v,
                    sems):
        wid = lax.axis_index("s") * NUM_CORES + lax.axis_index("c")
        base = wid * B_PER_W
        pltpu.sync_copy(x_hbm.at[pl.ds(base, B_PER_W)], idx_v)

        def gather(i, buf, sem):
            return pltpu.make_async_copy(table_hbm.at[idx_v.at[i]], buf, sem)

        def accumulate(i, buf):
            for c in range(EMBED_DIM // 16):
                sl = pl.ds(c * 16, 16)
                acc = buf[0, sl]
                for j in range(1, HIST):
                    acc = acc + buf[j, sl]
                acc_v[i, sl] = acc

        gather(0, rows_a, sems.at[0]).start()

        def pair_body(t, carry):
            i0 = 2 * t
            gather(i0 + 1, rows_b, sems.at[1]).start()
            gather(i0, rows_a, sems.at[0]).wait()
            accumulate(i0, rows_a)

            @pl.when(t + 1 < B_PER_W // 2)
            def _prefetch_next():
                gather(i0 + 2, rows_a, sems.at[0]).start()

            gather(i0 + 1, rows_b, sems.at[1]).wait()
            accumulate(i0 + 1, rows_b)
            return carry

        lax.fori_loop(0, B_PER_W // 2, pair_body, 0)
        pltpu.sync_copy(acc_v, out_hbm.at[pl.ds(base, B_PER_W)])

    table128 = jnp.pad(embed_weight, ((0, 0), (0, 128 - EMBED_DIM)))
    return pool_kernel(x, table128)


BN = 4096


def _mm_kernel(wt_ref, s_ref, o_ref):
    o_ref[...] = lax.dot_general(
        wt_ref[...], s_ref[...],
        dimension_numbers=(((0,), (1,)), ((), ())),
        preferred_element_type=jnp.float32,
    )


def _project_call(wt, s):
    grid = (VOCAB + BN - 1) // BN
    return pl.pallas_call(
        _mm_kernel,
        grid=(grid,),
        in_specs=[
            pl.BlockSpec((EMBED_DIM, BN), lambda j: (0, j)),
            pl.BlockSpec((BATCH, EMBED_DIM), lambda j: (0, 0)),
        ],
        out_specs=pl.BlockSpec((BN, BATCH), lambda j: (j, 0)),
        out_shape=jax.ShapeDtypeStruct((VOCAB, BATCH), jnp.float32),
    )(wt, s)


def kernel(x, embed_weight, linear_weight):
    x = x.astype(jnp.int32)
    pooled = _pool_call(x, embed_weight)
    out_t = _project_call(linear_weight.T, pooled)
    return out_t.T


# speedup vs baseline: 1.7570x; 1.0021x over previous
import functools

import jax
import jax.numpy as jnp
from jax import lax
from jax.experimental import pallas as pl
from jax.experimental.pallas import tpu as pltpu
from jax.experimental.pallas import tpu_sc as plsc

VOCAB = 100000
EMBED_DIM = 64
BATCH = 1024
HIST = 50

NUM_CORES = 2
NUM_SUBCORES = 16
NUM_WORKERS = NUM_CORES * NUM_SUBCORES
B_PER_W = BATCH // NUM_WORKERS


def _pool_call(x, embed_weight):
    mesh = plsc.VectorSubcoreMesh(core_axis_name="c", subcore_axis_name="s")

    @functools.partial(
        pl.kernel,
        mesh=mesh,

        out_type=jax.ShapeDtypeStruct((BATCH, EMBED_DIM), jnp.float32),
        scratch_types=[
            pltpu.VMEM((B_PER_W, HIST), jnp.int32),
            pltpu.VMEM((HIST, 128), jnp.float32),
            pltpu.VMEM((HIST, 128), jnp.float32),
            pltpu.VMEM((B_PER_W, EMBED_DIM), jnp.float32),
            pltpu.SemaphoreType.DMA((2,)),
        ],
    )
    def pool_kernel(x_hbm, table_hbm, out_hbm, idx_v, rows_a, rows_b, acc_v,
                    sems):
        wid = lax.axis_index("s") * NUM_CORES + lax.axis_index("c")
        base = wid * B_PER_W
        pltpu.sync_copy(x_hbm.at[pl.ds(base, B_PER_W)], idx_v)

        def gather(i, buf, sem):
            return pltpu.make_async_copy(table_hbm.at[idx_v.at[i]], buf, sem)

        def accumulate(i, buf):
            for c in range(EMBED_DIM // 16):
                sl = pl.ds(c * 16, 16)
                acc = buf[0, sl]
                for j in range(1, HIST):
                    acc = acc + buf[j, sl]
                acc_v[i, sl] = acc

        gather(0, rows_a, sems.at[0]).start()

        def pair_body(t, carry):
            i0 = 2 * t
            gather(i0 + 1, rows_b, sems.at[1]).start()
            gather(i0, rows_a, sems.at[0]).wait()
            accumulate(i0, rows_a)

            @pl.when(t + 1 < B_PER_W // 2)
            def _prefetch_next():
                gather(i0 + 2, rows_a, sems.at[0]).start()

            gather(i0 + 1, rows_b, sems.at[1]).wait()
            accumulate(i0 + 1, rows_b)
            return carry

        lax.fori_loop(0, B_PER_W // 2, pair_body, 0)
        pltpu.sync_copy(acc_v, out_hbm.at[pl.ds(base, B_PER_W)])

    table128 = jnp.pad(embed_weight, ((0, 0), (0, 128 - EMBED_DIM)))
    return pool_kernel(x, table128)


BN = 4096


def _mm_kernel(wt_ref, s_ref, o_ref):
    o_ref[...] = lax.dot_general(
        wt_ref[...], s_ref[...],
        dimension_numbers=(((0,), (1,)), ((), ())),
        preferred_element_type=jnp.float32,
    )


def _project_call(wt, s):
    grid = (VOCAB + BN - 1) // BN
    return pl.pallas_call(
        _mm_kernel,
        grid=(grid,),
        in_specs=[
            pl.BlockSpec((EMBED_DIM, BN), lambda j: (0, j)),
            pl.BlockSpec((BATCH, EMBED_DIM), lambda j: (0, 0)),
        ],
        out_specs=pl.BlockSpec((BN, BATCH), lambda j: (j, 0)),
        out_shape=jax.ShapeDtypeStruct((VOCAB, BATCH), jnp.float32),
        compiler_params=pltpu.CompilerParams(
            fuse_transposed_lhs_in_matmul=True,
        ),
    )(wt, s)


def kernel(x, embed_weight, linear_weight):
    x = x.astype(jnp.int32)
    pooled = _pool_call(x, embed_weight)
    out_t = _project_call(linear_weight.T, pooled)
    return out_t.T
